# baseline (speedup 1.0000x reference)
import functools

import jax
import jax.numpy as jnp
from jax import lax
from jax.experimental import pallas as pl
from jax.experimental.pallas import tpu as pltpu
from jax.experimental.pallas import tpu_sc as plsc

_EDIM = 32
_D = 64
_B = 16384
_NC = 2
_NS = 16
_NW = _NC * _NS
_BPW = _B // _NW
_CHUNK = 128
_NCHUNK = _BPW // _CHUNK
_BLK = 2048
_GRID = _B // _BLK


def _sc_gather_body(uid_hbm, iid_hbm, ut_hbm, it_hbm, ue_hbm, ie_hbm,
                    uidx_v, iidx_v, urows_v, irows_v, sem):
    wid = lax.axis_index("s") * _NC + lax.axis_index("c")
    base = wid * _BPW
    pltpu.sync_copy(uid_hbm.at[wid], uidx_v)
    pltpu.sync_copy(iid_hbm.at[wid], iidx_v)
    copies = []
    for j in range(_NCHUNK):
        copies.append(pltpu.async_copy(
            ut_hbm.at[uidx_v.at[j]], urows_v.at[pl.ds(j * _CHUNK, _CHUNK)], sem))
        copies.append(pltpu.async_copy(
            it_hbm.at[iidx_v.at[j]], irows_v.at[pl.ds(j * _CHUNK, _CHUNK)], sem))
    for c in copies:
        c.wait()
    pltpu.sync_copy(urows_v, ue_hbm.at[pl.ds(base, _BPW)])
    pltpu.sync_copy(irows_v, ie_hbm.at[pl.ds(base, _BPW)])


_sc_gather = functools.partial(
    pl.kernel,
    out_type=[
        jax.ShapeDtypeStruct((_B, _D), jnp.float32),
        jax.ShapeDtypeStruct((_B, _D), jnp.float32),
    ],
    mesh=plsc.VectorSubcoreMesh(core_axis_name="c", subcore_axis_name="s"),
    compiler_params=pltpu.CompilerParams(use_tc_tiling_on_sc=False),
    scratch_types=[
        pltpu.VMEM((_NCHUNK, _CHUNK), jnp.int32),
        pltpu.VMEM((_NCHUNK, _CHUNK), jnp.int32),
        pltpu.VMEM((_BPW, _D), jnp.float32),
        pltpu.VMEM((_BPW, _D), jnp.float32),
        pltpu.SemaphoreType.DMA,
    ],
)(_sc_gather_body)


def _mlp_body(ue_ref, ie_ref, w1u_ref, w1i_ref, b1_ref, w2_ref, b2_ref,
              w3l_ref, w3r_ref, b3_ref, o_ref):
    ue = ue_ref[...]
    ie = ie_ref[...]
    left = ue[:, :_EDIM] * ie[:, :_EDIM]
    h1 = jnp.dot(ue[:, _EDIM:], w1u_ref[...], preferred_element_type=jnp.float32)
    h1 = h1 + jnp.dot(ie[:, _EDIM:], w1i_ref[...], preferred_element_type=jnp.float32)
    h1 = jnp.maximum(h1 + b1_ref[...], 0.0)
    h2 = jnp.dot(h1, w2_ref[...], preferred_element_type=jnp.float32)
    h2 = jnp.maximum(h2 + b2_ref[...], 0.0)
    o = jnp.sum(left * w3l_ref[...], axis=1) + jnp.sum(h2 * w3r_ref[...], axis=1)
    o_ref[...] = o + b3_ref[...]


def _mlp_head(ue, ie, w1u, w1i, b1, w2, b2, w3l, w3r, b3):
    full = lambda shape: pl.BlockSpec(shape, lambda i: (0, 0))
    return pl.pallas_call(
        _mlp_body,
        grid=(_GRID,),
        in_specs=[
            pl.BlockSpec((_BLK, _D), lambda i: (i, 0)),
            pl.BlockSpec((_BLK, _D), lambda i: (i, 0)),
            full((_EDIM, _EDIM)),
            full((_EDIM, _EDIM)),
            full((1, _EDIM)),
            full((_EDIM, _EDIM // 2)),
            full((1, _EDIM // 2)),
            full((1, _EDIM)),
            full((1, _EDIM // 2)),
            pl.BlockSpec((1,), lambda i: (0,)),
        ],
        out_specs=pl.BlockSpec((_BLK,), lambda i: (i,)),
        out_shape=jax.ShapeDtypeStruct((_B,), jnp.float32),
    )(ue, ie, w1u, w1i, b1, w2, b2, w3l, w3r, b3)


def kernel(user_ids, item_ids, user_table, item_table, W1, b1, W2, b2, W3, b3):
    uid = user_ids.astype(jnp.int32).reshape(_NW, _NCHUNK, _CHUNK)
    iid = item_ids.astype(jnp.int32).reshape(_NW, _NCHUNK, _CHUNK)
    ue, ie = _sc_gather(uid, iid, user_table, item_table)
    return _mlp_head(
        ue, ie,
        W1[:, :_EDIM].T, W1[:, _EDIM:].T, b1[None, :],
        W2.T, b2[None, :],
        W3[:, :_EDIM], W3[:, _EDIM:], b3,
    )


# speedup vs baseline: 2.3823x; 2.3823x over previous
import functools

import jax
import jax.numpy as jnp
from jax import lax
from jax.experimental import pallas as pl
from jax.experimental.pallas import tpu as pltpu
from jax.experimental.pallas import tpu_sc as plsc

_EDIM = 32
_D = 64
_B = 16384
_NROW = 1000000
_NC = 2
_NS = 16
_NW = _NC * _NS
_BPW = _B // _NW
_UNROLL = 16


def _sc_gather_body(uid_hbm, iid_hbm, ut_hbm, it_hbm, ue_hbm, ie_hbm,
                    ids_v, sel_v, sem):
    wid = lax.axis_index("s") * _NC + lax.axis_index("c")
    base = wid * _BPW

    for ids, table, out in ((uid_hbm, ut_hbm, ue_hbm),
                            (iid_hbm, it_hbm, ie_hbm)):
        pltpu.sync_copy(ids.at[wid], ids_v)

        def fire(g, _):
            vec = ids_v[pl.ds(pl.multiple_of(g * _UNROLL, _UNROLL), _UNROLL)]
            for u in range(_UNROLL):
                rid = vec[u]
                pltpu.async_copy(
                    table.at[rid >> 3, rid & 7],
                    sel_v.at[g * _UNROLL + u], sem)
            return _

        lax.fori_loop(0, _BPW // _UNROLL, fire, None)

        def drain(g, _):
            for _u in range(_UNROLL):
                pltpu.make_async_copy(table.at[0, 0], sel_v.at[0], sem).wait()
            return _

        lax.fori_loop(0, _BPW // _UNROLL, drain, None)
        pltpu.sync_copy(sel_v, out.at[pl.ds(base, _BPW)])


_sc_gather = functools.partial(
    pl.kernel,
    out_type=[
        jax.ShapeDtypeStruct((_B, _D), jnp.float32),
        jax.ShapeDtypeStruct((_B, _D), jnp.float32),
    ],
    mesh=plsc.VectorSubcoreMesh(core_axis_name="c", subcore_axis_name="s"),
    scratch_types=[
        pltpu.VMEM((_BPW,), jnp.int32),
        pltpu.VMEM((_BPW, _D), jnp.float32),
        pltpu.SemaphoreType.DMA,
    ],
)(_sc_gather_body)


def _mlp_body(ue_ref, ie_ref, w1u_ref, w1i_ref, b1_ref, w2_ref, b2_ref,
              w3l_ref, w3r_ref, b3_ref, o_ref):
    ue = ue_ref[...]
    ie = ie_ref[...]
    left = ue[:, :_EDIM] * ie[:, :_EDIM]
    h1 = jnp.dot(ue[:, _EDIM:], w1u_ref[...], preferred_element_type=jnp.float32)
    h1 = h1 + jnp.dot(ie[:, _EDIM:], w1i_ref[...], preferred_element_type=jnp.float32)
    h1 = jnp.maximum(h1 + b1_ref[...], 0.0)
    h2 = jnp.dot(h1, w2_ref[...], preferred_element_type=jnp.float32)
    h2 = jnp.maximum(h2 + b2_ref[...], 0.0)
    o = jnp.sum(left * w3l_ref[...], axis=1) + jnp.sum(h2 * w3r_ref[...], axis=1)
    o_ref[...] = o + b3_ref[...]


_BLK = 2048
_GRID = _B // _BLK


def _mlp_head(ue, ie, w1u, w1i, b1, w2, b2, w3l, w3r, b3):
    full = lambda shape: pl.BlockSpec(shape, lambda i: (0, 0))
    return pl.pallas_call(
        _mlp_body,
        grid=(_GRID,),
        in_specs=[
            pl.BlockSpec((_BLK, _D), lambda i: (i, 0)),
            pl.BlockSpec((_BLK, _D), lambda i: (i, 0)),
            full((_EDIM, _EDIM)),
            full((_EDIM, _EDIM)),
            full((1, _EDIM)),
            full((_EDIM, _EDIM // 2)),
            full((1, _EDIM // 2)),
            full((1, _EDIM)),
            full((1, _EDIM // 2)),
            pl.BlockSpec((1,), lambda i: (0,)),
        ],
        out_specs=pl.BlockSpec((_BLK,), lambda i: (i,)),
        out_shape=jax.ShapeDtypeStruct((_B,), jnp.float32),
    )(ue, ie, w1u, w1i, b1, w2, b2, w3l, w3r, b3)


def kernel(user_ids, item_ids, user_table, item_table, W1, b1, W2, b2, W3, b3):
    uid = user_ids.astype(jnp.int32).reshape(_NW, _BPW)
    iid = item_ids.astype(jnp.int32).reshape(_NW, _BPW)
    ut3 = user_table.reshape(_NROW // 8, 8, _D)
    it3 = item_table.reshape(_NROW // 8, 8, _D)
    ue, ie = _sc_gather(uid, iid, ut3, it3)
    return _mlp_head(
        ue, ie,
        W1[:, :_EDIM].T, W1[:, _EDIM:].T, b1[None, :],
        W2.T, b2[None, :],
        W3[:, :_EDIM], W3[:, _EDIM:], b3,
    )
